# baseline (speedup 1.0000x reference)
import functools
import jax
import jax.numpy as jnp
from jax.experimental import pallas as pl
from jax.experimental.pallas import tpu as pltpu

N = 10000
E = 160000
DF = 128
HID = 64
HEADS = 4
NCLS = 16
G = 512


def _gatv2(x, edge_index, Wl, bl, Wr, br, att, bo, heads, oc, self_loops):
    n = x.shape[0]
    if self_loops:
        loop = jnp.arange(n, dtype=edge_index.dtype)
        edge_index = jnp.concatenate([edge_index, jnp.stack([loop, loop])], axis=1)
    src = edge_index[0]
    dst = edge_index[1]
    xl = (x @ Wl + bl).reshape(n, heads, oc)
    xr = (x @ Wr + br).reshape(n, heads, oc)
    m = xl[src] + xr[dst]
    e = jnp.sum(jax.nn.leaky_relu(m, 0.2) * att[None, :, :], axis=-1)
    emax = jax.ops.segment_max(e, dst, num_segments=n)
    emax = jnp.where(jnp.isfinite(emax), emax, 0.0)
    ex = jnp.exp(e - emax[dst])
    denom = jax.ops.segment_sum(ex, dst, num_segments=n)
    alpha = ex / jnp.maximum(denom[dst], 1e-16)
    out = jax.ops.segment_sum(alpha[:, :, None] * xl[src], dst, num_segments=n)
    return out.reshape(n, heads * oc) + bo


def _mlp_body(pooled_ref, W1_ref, b1_ref, W2_ref, b2_ref, W3_ref, b3_ref, out_ref):
    h = jnp.maximum(jnp.dot(pooled_ref[...], W1_ref[...],
                            preferred_element_type=jnp.float32) + b1_ref[...], 0.0)
    h = jnp.maximum(jnp.dot(h, W2_ref[...],
                            preferred_element_type=jnp.float32) + b2_ref[...], 0.0)
    logits = jnp.dot(h, W3_ref[...], preferred_element_type=jnp.float32) + b3_ref[...]
    mx = jnp.max(logits, axis=1, keepdims=True)
    ex = jnp.exp(logits - mx)
    out_ref[...] = ex / jnp.sum(ex, axis=1, keepdims=True)


def kernel(x, edge_index, batch, Wl1, bl1, Wr1, br1, att1, bo1, Wl2, bl2, Wr2, br2,
           att2, bo2, Wl3, bl3, Wr3, br3, att3, bo3, W1, b1, W2, b2, W3, b3):
    h = jax.nn.relu(_gatv2(x, edge_index, Wl1, bl1, Wr1, br1, att1, bo1, HEADS, HID, False))
    h = jax.nn.relu(_gatv2(h, edge_index, Wl2, bl2, Wr2, br2, att2, bo2, HEADS, HID, True))
    h = _gatv2(h, edge_index, Wl3, bl3, Wr3, br3, att3, bo3, 1, HID, True)
    sums = jax.ops.segment_sum(h, batch, num_segments=G)
    cnt = jax.ops.segment_sum(jnp.ones((h.shape[0],), jnp.float32), batch, num_segments=G)
    pooled = sums / jnp.maximum(cnt, 1.0)[:, None]

    out = pl.pallas_call(
        _mlp_body,
        out_shape=jax.ShapeDtypeStruct((G, NCLS), jnp.float32),
    )(pooled, W1, b1.reshape(1, HID), W2, b2.reshape(1, HID), W3, b3.reshape(1, NCLS))
    return out


# speedup vs baseline: 12.7748x; 12.7748x over previous
import functools
import jax
import jax.numpy as jnp
from jax import lax
from jax.experimental import pallas as pl
from jax.experimental.pallas import tpu as pltpu
from jax.experimental.pallas import tpu_sc as plsc

N = 10000
DF = 128
HID = 64
HEADS = 4
NCLS = 16
G = 512

NP = 10240
RPT = NP // 16
C = 64
NEG = -3.4e38
EPS = 1e-30

E1 = 160000
E2 = 170000


def _pad_to(e, g):
    return ((e + g - 1) // g) * g


EP1 = _pad_to(E1, 16 * C)
EP2 = _pad_to(E2, 32 * C)

_mesh = plsc.VectorSubcoreMesh(core_axis_name="c", subcore_axis_name="s")
_sc_params = pltpu.CompilerParams(needs_layout_passes=False)


def _f32(shape):
    return jax.ShapeDtypeStruct(shape, jnp.float32)


def _reduce_rows(gm_b, nrows):
    m = gm_b[0]
    for r in range(1, nrows):
        m = jnp.maximum(m, gm_b[r])
    return jnp.full((16,), jnp.max(m), jnp.float32)


def _sc_edge_scores_cs(EP):
    nchunks = EP // 16 // C

    @functools.partial(
        pl.kernel,
        out_type=(_f32((4, EP)), _f32((4, 16, 16))),
        mesh=_mesh,
        compiler_params=_sc_params,
        scratch_types=[
            pltpu.VMEM((C,), jnp.int32),
            pltpu.VMEM((C,), jnp.int32),
            pltpu.VMEM((C, 128), jnp.float32),
            pltpu.VMEM((C, 128), jnp.float32),
            pltpu.VMEM((2, 64), jnp.float32),
            pltpu.VMEM((2, C), jnp.float32),
            pltpu.VMEM((2, 16), jnp.float32),
            pltpu.SemaphoreType.DMA,
            pltpu.SemaphoreType.DMA,
        ],
    )
    def body(xl_tab, xr_tab, src_h, dst_h, att_h, e_out, gmax_out,
             src_b, dst_b, rl, rr, attv, e_b, maxb, sem1, sem2):
        c = lax.axis_index("c")
        s = lax.axis_index("s")
        tbase = s * (EP // 16)
        coffv = jnp.full((16,), c * NP, jnp.int32)
        pltpu.sync_copy(att_h.at[pl.ds(2 * c, 2)], attv)
        maxb[0] = jnp.full((16,), NEG, jnp.float32)
        maxb[1] = jnp.full((16,), NEG, jnp.float32)
        att_regs = [[attv[h, pl.ds(16 * q, 16)] for q in range(4)]
                    for h in range(2)]
        iot = jnp.arange(16, dtype=jnp.int32)

        def chunk(j, carry):
            base = tbase + j * C
            pltpu.sync_copy(src_h.at[pl.ds(base, C)], src_b)
            pltpu.sync_copy(dst_h.at[pl.ds(base, C)], dst_b)
            for q in range(4):
                src_b[pl.ds(16 * q, 16)] = src_b[pl.ds(16 * q, 16)] + coffv
                dst_b[pl.ds(16 * q, 16)] = dst_b[pl.ds(16 * q, 16)] + coffv
            d1 = pltpu.async_copy(xl_tab.at[src_b], rl, sem1)
            d2 = pltpu.async_copy(xr_tab.at[dst_b], rr, sem2)
            d1.wait()
            d2.wait()
            for g in range(C // 16):
                evs = [jnp.zeros((16,), jnp.float32) for _ in range(2)]
                for r in range(16):
                    i = 16 * g + r
                    for h in range(2):
                        acc = None
                        for q in range(4):
                            v = 4 * h + q
                            a = (rl[i, pl.ds(16 * v, 16)] +
                                 rr[i, pl.ds(16 * v, 16)])
                            t = (0.6 * a + 0.4 * jnp.abs(a)) * att_regs[h][q]
                            acc = t if q == 0 else acc + t
                        evs[h] = jnp.where(iot == r, jnp.sum(acc), evs[h])
                for h in range(2):
                    e_b[h, pl.ds(16 * g, 16)] = evs[h]
                    maxb[h] = jnp.maximum(maxb[h], evs[h])
            for h in range(2):
                pltpu.sync_copy(e_b.at[h], e_out.at[2 * c + h, pl.ds(base, C)])
            return carry

        lax.fori_loop(0, nchunks, chunk, 0)
        for h in range(2):
            pltpu.sync_copy(maxb.at[h], gmax_out.at[2 * c + h, s])

    return body


def _sc_denom_cs(EP):
    nchunks = EP // 16 // C

    @functools.partial(
        pl.kernel,
        out_type=_f32((2 * NP, 128)),
        mesh=_mesh,
        compiler_params=_sc_params,
        scratch_types=[
            pltpu.VMEM_SHARED((NP, 128), jnp.float32),
            pltpu.VMEM((C,), jnp.int32),
            pltpu.VMEM((2, C), jnp.float32),
            pltpu.VMEM((2, C), jnp.float32),
            pltpu.VMEM((C, 128), jnp.float32),
            pltpu.VMEM((16, 16), jnp.float32),
        ],
    )
    def body(e_h, gmax_h, dst_h, den_out, den_sp, dst_b, e_b, a_b, stage,
             gm_b):
        c = lax.axis_index("c")
        s = lax.axis_index("s")
        gms = []
        for h in range(2):
            pltpu.sync_copy(gmax_h.at[2 * c + h], gm_b)
            gms.append(_reduce_rows(gm_b, 16))
        for i in range(C):
            for v in range(8):
                stage[i, pl.ds(16 * v, 16)] = jnp.zeros((16,), jnp.float32)
        for k in range(RPT // C):
            pltpu.sync_copy(stage, den_sp.at[pl.ds(RPT * s + C * k, C)])
        plsc.subcore_barrier()
        iot = jnp.arange(16, dtype=jnp.int32)
        m0 = (iot == 0).astype(jnp.float32)
        m1 = (iot == 1).astype(jnp.float32)

        def chunk(j, carry):
            base = s * (EP // 16) + j * C
            pltpu.sync_copy(dst_h.at[pl.ds(base, C)], dst_b)
            for h in range(2):
                pltpu.sync_copy(e_h.at[2 * c + h, pl.ds(base, C)], e_b.at[h])
                for q in range(4):
                    a_b[h, pl.ds(16 * q, 16)] = jnp.exp(
                        e_b[h, pl.ds(16 * q, 16)] - gms[h])
            for g in range(C // 16):
                av0 = a_b[0, pl.ds(16 * g, 16)]
                av1 = a_b[1, pl.ds(16 * g, 16)]
                for r in range(16):
                    i = 16 * g + r
                    stage[i, pl.ds(0, 16)] = (
                        jnp.full((16,), av0[r], jnp.float32) * m0 +
                        jnp.full((16,), av1[r], jnp.float32) * m1)
            pltpu.sync_copy(stage, den_sp.at[dst_b], add=True)
            return carry

        lax.fori_loop(0, nchunks, chunk, 0)
        plsc.subcore_barrier()
        pltpu.sync_copy(den_sp.at[pl.ds(RPT * s, RPT)],
                        den_out.at[pl.ds(c * NP + RPT * s, RPT)])

    return body


def _sc_aggregate_cs(EP):
    nchunks = EP // 16 // C

    @functools.partial(
        pl.kernel,
        out_type=_f32((2 * NP, 128)),
        mesh=_mesh,
        compiler_params=_sc_params,
        scratch_types=[
            pltpu.VMEM_SHARED((NP, 128), jnp.float32),
            pltpu.VMEM((C,), jnp.int32),
            pltpu.VMEM((C,), jnp.int32),
            pltpu.VMEM((C, 128), jnp.float32),
            pltpu.VMEM((C, 128), jnp.float32),
            pltpu.VMEM((C, 128), jnp.float32),
            pltpu.VMEM((2, C), jnp.float32),
            pltpu.VMEM((2, C), jnp.float32),
            pltpu.VMEM((16, 16), jnp.float32),
            pltpu.SemaphoreType.DMA,
            pltpu.SemaphoreType.DMA,
        ],
    )
    def body(xl_tab, src_h, dst_h, e_h, den_h, gmax_h, agg_out,
             out_sp, src_b, dst_b, rl, stage, db, e_b, a_b, gm_b,
             sem1, sem2):
        c = lax.axis_index("c")
        s = lax.axis_index("s")
        coffv = jnp.full((16,), c * NP, jnp.int32)
        gms = []
        for h in range(2):
            pltpu.sync_copy(gmax_h.at[2 * c + h], gm_b)
            gms.append(_reduce_rows(gm_b, 16))
        for i in range(C):
            for v in range(8):
                stage[i, pl.ds(16 * v, 16)] = jnp.zeros((16,), jnp.float32)
        for k in range(RPT // C):
            pltpu.sync_copy(stage, out_sp.at[pl.ds(RPT * s + C * k, C)])
        plsc.subcore_barrier()

        def chunk(j, carry):
            base = s * (EP // 16) + j * C
            pltpu.sync_copy(src_h.at[pl.ds(base, C)], src_b)
            pltpu.sync_copy(dst_h.at[pl.ds(base, C)], dst_b)
            for q in range(4):
                src_b[pl.ds(16 * q, 16)] = src_b[pl.ds(16 * q, 16)] + coffv
            d1 = pltpu.async_copy(xl_tab.at[src_b], rl, sem1)
            for h in range(2):
                pltpu.sync_copy(e_h.at[2 * c + h, pl.ds(base, C)], e_b.at[h])
                for q in range(4):
                    a_b[h, pl.ds(16 * q, 16)] = jnp.exp(
                        e_b[h, pl.ds(16 * q, 16)] - gms[h])
            d1.wait()
            for g in range(C // 16):
                av0 = a_b[0, pl.ds(16 * g, 16)]
                av1 = a_b[1, pl.ds(16 * g, 16)]
                for r in range(16):
                    i = 16 * g + r
                    a0 = jnp.full((16,), av0[r], jnp.float32)
                    a1 = jnp.full((16,), av1[r], jnp.float32)
                    for v in range(8):
                        stage[i, pl.ds(16 * v, 16)] = (
                            rl[i, pl.ds(16 * v, 16)] * (a0 if v < 4 else a1))
            pltpu.sync_copy(stage, out_sp.at[dst_b], add=True)
            return carry

        lax.fori_loop(0, nchunks, chunk, 0)
        plsc.subcore_barrier()
        for k in range(RPT // C):
            rowbase = RPT * s + C * k
            pltpu.sync_copy(out_sp.at[pl.ds(rowbase, C)], rl)
            pltpu.sync_copy(den_h.at[pl.ds(c * NP + rowbase, C)], db)
            for i in range(C):
                rv = 1.0 / jnp.maximum(db[i, pl.ds(0, 16)], EPS)
                r0 = jnp.full((16,), rv[0], jnp.float32)
                r1 = jnp.full((16,), rv[1], jnp.float32)
                for v in range(8):
                    stage[i, pl.ds(16 * v, 16)] = (
                        rl[i, pl.ds(16 * v, 16)] * (r0 if v < 4 else r1))
            pltpu.sync_copy(stage, agg_out.at[pl.ds(c * NP + rowbase, C)])

    return body


def _sc_edge_scores_l3(EP):
    span = EP // 32
    nchunks = span // C

    @functools.partial(
        pl.kernel,
        out_type=(_f32((EP,)), _f32((32, 16))),
        mesh=_mesh,
        compiler_params=_sc_params,
        scratch_types=[
            pltpu.VMEM((C,), jnp.int32),
            pltpu.VMEM((C,), jnp.int32),
            pltpu.VMEM((C, 128), jnp.float32),
            pltpu.VMEM((C, 128), jnp.float32),
            pltpu.VMEM((1, 64), jnp.float32),
            pltpu.VMEM((C,), jnp.float32),
            pltpu.VMEM((16,), jnp.float32),
            pltpu.SemaphoreType.DMA,
            pltpu.SemaphoreType.DMA,
        ],
    )
    def body(xl_tab, xr_tab, src_h, dst_h, att_h, e_out, gmax_out,
             src_b, dst_b, rl, rr, attv, e_b, maxb, sem1, sem2):
        c = lax.axis_index("c")
        s = lax.axis_index("s")
        wid = c * 16 + s
        tbase = wid * span
        pltpu.sync_copy(att_h, attv)
        maxb[...] = jnp.full((16,), NEG, jnp.float32)
        att_regs = [attv[0, pl.ds(16 * q, 16)] for q in range(4)]
        iot = jnp.arange(16, dtype=jnp.int32)

        def chunk(j, carry):
            base = tbase + j * C
            pltpu.sync_copy(src_h.at[pl.ds(base, C)], src_b)
            pltpu.sync_copy(dst_h.at[pl.ds(base, C)], dst_b)
            d1 = pltpu.async_copy(xl_tab.at[src_b], rl, sem1)
            d2 = pltpu.async_copy(xr_tab.at[dst_b], rr, sem2)
            d1.wait()
            d2.wait()
            for g in range(C // 16):
                ev = jnp.zeros((16,), jnp.float32)
                for r in range(16):
                    i = 16 * g + r
                    acc = None
                    for q in range(4):
                        a = rl[i, pl.ds(16 * q, 16)] + rr[i, pl.ds(16 * q, 16)]
                        t = (0.6 * a + 0.4 * jnp.abs(a)) * att_regs[q]
                        acc = t if q == 0 else acc + t
                    ev = jnp.where(iot == r, jnp.sum(acc), ev)
                e_b[pl.ds(16 * g, 16)] = ev
                maxb[...] = jnp.maximum(maxb[...], ev)
            pltpu.sync_copy(e_b, e_out.at[pl.ds(base, C)])
            return carry

        lax.fori_loop(0, nchunks, chunk, 0)
        pltpu.sync_copy(maxb, gmax_out.at[wid])

    return body


def _sc_denom_l3(EP):
    nchunks = EP // 16 // C

    @functools.partial(
        pl.kernel,
        out_type=_f32((NP, 128)),
        mesh=_mesh,
        compiler_params=_sc_params,
        scratch_types=[
            pltpu.VMEM_SHARED((NP, 128), jnp.float32),
            pltpu.VMEM((C,), jnp.int32),
            pltpu.VMEM((C,), jnp.float32),
            pltpu.VMEM((C,), jnp.float32),
            pltpu.VMEM((C, 128), jnp.float32),
            pltpu.VMEM((32, 16), jnp.float32),
        ],
    )
    def body(e_h, gmax_h, dst_h, den_out, den_sp, dst_b, e_b, a_b, stage,
             gm_b):
        c = lax.axis_index("c")
        s = lax.axis_index("s")
        pltpu.sync_copy(gmax_h, gm_b)
        gm = _reduce_rows(gm_b, 32)
        for i in range(C):
            for v in range(8):
                stage[i, pl.ds(16 * v, 16)] = jnp.zeros((16,), jnp.float32)
        for k in range(RPT // C):
            pltpu.sync_copy(stage, den_sp.at[pl.ds(RPT * s + C * k, C)])
        plsc.subcore_barrier()
        iot = jnp.arange(16, dtype=jnp.int32)
        m0 = (iot == 0).astype(jnp.float32)

        def chunk(j, carry):
            base = s * (EP // 16) + j * C
            pltpu.sync_copy(dst_h.at[pl.ds(base, C)], dst_b)
            pltpu.sync_copy(e_h.at[pl.ds(base, C)], e_b)
            for q in range(4):
                a_b[pl.ds(16 * q, 16)] = jnp.exp(e_b[pl.ds(16 * q, 16)] - gm)
            for g in range(C // 16):
                av = a_b[pl.ds(16 * g, 16)]
                for r in range(16):
                    i = 16 * g + r
                    stage[i, pl.ds(0, 16)] = (
                        jnp.full((16,), av[r], jnp.float32) * m0)
            pltpu.sync_copy(stage, den_sp.at[dst_b], add=True)
            return carry

        lax.fori_loop(0, nchunks, chunk, 0)
        plsc.subcore_barrier()

        @pl.when(c == 0)
        def _():
            pltpu.sync_copy(den_sp.at[pl.ds(RPT * s, RPT)],
                            den_out.at[pl.ds(RPT * s, RPT)])

    return body


def _sc_aggregate_l3(EP):
    span = EP // 32
    nchunks = span // C

    @functools.partial(
        pl.kernel,
        out_type=_f32((2 * NP, 128)),
        mesh=_mesh,
        compiler_params=_sc_params,
        scratch_types=[
            pltpu.VMEM_SHARED((NP, 128), jnp.float32),
            pltpu.VMEM((C,), jnp.int32),
            pltpu.VMEM((C,), jnp.int32),
            pltpu.VMEM((C, 128), jnp.float32),
            pltpu.VMEM((C, 128), jnp.float32),
            pltpu.VMEM((C, 128), jnp.float32),
            pltpu.VMEM((C,), jnp.float32),
            pltpu.VMEM((C,), jnp.float32),
            pltpu.VMEM((32, 16), jnp.float32),
            pltpu.SemaphoreType.DMA,
            pltpu.SemaphoreType.DMA,
        ],
    )
    def body(xl_tab, src_h, dst_h, e_h, den_h, gmax_h, agg_out,
             out_sp, src_b, dst_b, rl, stage, db, e_b, a_b, gm_b, sem1, sem2):
        c = lax.axis_index("c")
        s = lax.axis_index("s")
        wid = c * 16 + s
        tbase = wid * span
        pltpu.sync_copy(gmax_h, gm_b)
        gm = _reduce_rows(gm_b, 32)
        for i in range(C):
            for v in range(8):
                stage[i, pl.ds(16 * v, 16)] = jnp.zeros((16,), jnp.float32)
        for k in range(RPT // C):
            pltpu.sync_copy(stage, out_sp.at[pl.ds(RPT * s + C * k, C)])
        plsc.subcore_barrier()

        def chunk(j, carry):
            base = tbase + j * C
            pltpu.sync_copy(src_h.at[pl.ds(base, C)], src_b)
            pltpu.sync_copy(dst_h.at[pl.ds(base, C)], dst_b)
            d1 = pltpu.async_copy(xl_tab.at[src_b], rl, sem1)
            pltpu.sync_copy(e_h.at[pl.ds(base, C)], e_b)
            for q in range(4):
                a_b[pl.ds(16 * q, 16)] = jnp.exp(e_b[pl.ds(16 * q, 16)] - gm)
            d1.wait()
            for g in range(C // 16):
                av = a_b[pl.ds(16 * g, 16)]
                for r in range(16):
                    i = 16 * g + r
                    a0 = jnp.full((16,), av[r], jnp.float32)
                    for v in range(4):
                        stage[i, pl.ds(16 * v, 16)] = (
                            rl[i, pl.ds(16 * v, 16)] * a0)
            pltpu.sync_copy(stage, out_sp.at[dst_b], add=True)
            return carry

        lax.fori_loop(0, nchunks, chunk, 0)
        plsc.subcore_barrier()
        for k in range(RPT // C):
            rowbase = RPT * s + C * k
            pltpu.sync_copy(out_sp.at[pl.ds(rowbase, C)], rl)
            pltpu.sync_copy(den_h.at[pl.ds(rowbase, C)], db)
            for i in range(C):
                rv = 1.0 / jnp.maximum(db[i, pl.ds(0, 16)], EPS)
                r0 = jnp.full((16,), rv[0], jnp.float32)
                for v in range(4):
                    stage[i, pl.ds(16 * v, 16)] = rl[i, pl.ds(16 * v, 16)] * r0
            pltpu.sync_copy(stage, agg_out.at[pl.ds(c * NP + rowbase, C)])

    return body


BR = 2560
NB = NP // BR


def _tc_proj1(x, Wl, Wr, bl, br):
    def body(x_ref, wl_ref, wr_ref, bl_ref, br_ref, xl_ref, xr_ref):
        a = x_ref[...]
        xl_ref[...] = jnp.dot(a, wl_ref[...],
                              preferred_element_type=jnp.float32) + bl_ref[...]
        xr_ref[...] = jnp.dot(a, wr_ref[...],
                              preferred_element_type=jnp.float32) + br_ref[...]

    return pl.pallas_call(
        body,
        grid=(2, NB),
        in_specs=[
            pl.BlockSpec((BR, 128), lambda g, b: (b, 0)),
            pl.BlockSpec((128, 128), lambda g, b: (0, g)),
            pl.BlockSpec((128, 128), lambda g, b: (0, g)),
            pl.BlockSpec((1, 128), lambda g, b: (0, g)),
            pl.BlockSpec((1, 128), lambda g, b: (0, g)),
        ],
        out_specs=[
            pl.BlockSpec((BR, 128), lambda g, b: (g * NB + b, 0)),
            pl.BlockSpec((BR, 128), lambda g, b: (g * NB + b, 0)),
        ],
        out_shape=[_f32((2 * NP, 128)), _f32((2 * NP, 128))],
    )(x, Wl, Wr, bl, br)


def _tc_proj_mid(agg, bo, Wl, Wr, bl, br):
    def body(lo_ref, hi_ref, bolo_ref, bohi_ref, wl_ref, wr_ref, bl_ref,
             br_ref, xl_ref, xr_ref):
        a = jnp.concatenate(
            [jnp.maximum(lo_ref[...] + bolo_ref[...], 0.0),
             jnp.maximum(hi_ref[...] + bohi_ref[...], 0.0)], axis=1)
        xl_ref[...] = jnp.dot(a, wl_ref[...],
                              preferred_element_type=jnp.float32) + bl_ref[...]
        xr_ref[...] = jnp.dot(a, wr_ref[...],
                              preferred_element_type=jnp.float32) + br_ref[...]

    return pl.pallas_call(
        body,
        grid=(2, NB),
        in_specs=[
            pl.BlockSpec((BR, 128), lambda g, b: (b, 0)),
            pl.BlockSpec((BR, 128), lambda g, b: (NB + b, 0)),
            pl.BlockSpec((1, 128), lambda g, b: (0, 0)),
            pl.BlockSpec((1, 128), lambda g, b: (0, 1)),
            pl.BlockSpec((256, 128), lambda g, b: (0, g)),
            pl.BlockSpec((256, 128), lambda g, b: (0, g)),
            pl.BlockSpec((1, 128), lambda g, b: (0, g)),
            pl.BlockSpec((1, 128), lambda g, b: (0, g)),
        ],
        out_specs=[
            pl.BlockSpec((BR, 128), lambda g, b: (g * NB + b, 0)),
            pl.BlockSpec((BR, 128), lambda g, b: (g * NB + b, 0)),
        ],
        out_shape=[_f32((2 * NP, 128)), _f32((2 * NP, 128))],
    )(agg, agg, bo, bo, Wl, Wr, bl, br)


def _tc_proj3(agg, bo, Wl, Wr, bl, br):
    def body(lo_ref, hi_ref, bolo_ref, bohi_ref, wl_ref, wr_ref, bl_ref,
             br_ref, xl_ref, xr_ref):
        a = jnp.concatenate(
            [jnp.maximum(lo_ref[...] + bolo_ref[...], 0.0),
             jnp.maximum(hi_ref[...] + bohi_ref[...], 0.0)], axis=1)
        xl_ref[...] = jnp.dot(a, wl_ref[...],
                              preferred_element_type=jnp.float32) + bl_ref[...]
        xr_ref[...] = jnp.dot(a, wr_ref[...],
                              preferred_element_type=jnp.float32) + br_ref[...]

    return pl.pallas_call(
        body,
        grid=(NB,),
        in_specs=[
            pl.BlockSpec((BR, 128), lambda b: (b, 0)),
            pl.BlockSpec((BR, 128), lambda b: (NB + b, 0)),
            pl.BlockSpec((1, 128), lambda b: (0, 0)),
            pl.BlockSpec((1, 128), lambda b: (0, 1)),
            pl.BlockSpec((256, 128), lambda b: (0, 0)),
            pl.BlockSpec((256, 128), lambda b: (0, 0)),
            pl.BlockSpec((1, 128), lambda b: (0, 0)),
            pl.BlockSpec((1, 128), lambda b: (0, 0)),
        ],
        out_specs=[
            pl.BlockSpec((BR, 128), lambda b: (b, 0)),
            pl.BlockSpec((BR, 128), lambda b: (b, 0)),
        ],
        out_shape=[_f32((NP, 128)), _f32((NP, 128))],
    )(agg, agg, bo, bo, Wl, Wr, bl, br)


def _tc_pool_mlp(agg3, batch3d, bo3, W1, b1, W2, b2, W3, b3):
    def body(b_ref, p0_ref, p1_ref, bo3_ref, w1_ref, b1_ref, w2_ref, b2_ref,
             w3_ref, b3_ref, out_ref, acc):
        step = pl.program_id(0)

        @pl.when(step == 0)
        def _():
            acc[...] = jnp.zeros((G, 72), jnp.float32)

        ids = b_ref[0, 0, :]
        sel = (ids[None, :] ==
               lax.broadcasted_iota(jnp.int32, (G, BR), 0)).astype(jnp.float32)
        h = (p0_ref[...][:, :64] + p1_ref[...][:, :64]) + bo3_ref[...]
        haug = jnp.concatenate(
            [h, jnp.ones((BR, 1), jnp.float32),
             jnp.zeros((BR, 7), jnp.float32)], axis=1)
        acc[...] += jnp.dot(sel, haug, preferred_element_type=jnp.float32)

        @pl.when(step == NB - 1)
        def _():
            a = acc[...]
            pooled = a[:, :64] / jnp.maximum(a[:, 64:65], 1.0)
            hh = jnp.maximum(
                jnp.dot(pooled, w1_ref[...],
                        preferred_element_type=jnp.float32) + b1_ref[...], 0.0)
            hh = jnp.maximum(
                jnp.dot(hh, w2_ref[...],
                        preferred_element_type=jnp.float32) + b2_ref[...], 0.0)
            logits = jnp.dot(hh, w3_ref[...],
                             preferred_element_type=jnp.float32) + b3_ref[...]
            mx = jnp.max(logits, axis=1, keepdims=True)
            ex = jnp.exp(logits - mx)
            out_ref[...] = ex / jnp.sum(ex, axis=1, keepdims=True)

    return pl.pallas_call(
        body,
        grid=(NB,),
        in_specs=[
            pl.BlockSpec((1, 1, BR), lambda b: (b, 0, 0)),
            pl.BlockSpec((BR, 128), lambda b: (b, 0)),
            pl.BlockSpec((BR, 128), lambda b: (NB + b, 0)),
            pl.BlockSpec((1, 64), lambda b: (0, 0)),
            pl.BlockSpec((64, 64), lambda b: (0, 0)),
            pl.BlockSpec((1, 64), lambda b: (0, 0)),
            pl.BlockSpec((64, 64), lambda b: (0, 0)),
            pl.BlockSpec((1, 64), lambda b: (0, 0)),
            pl.BlockSpec((64, 16), lambda b: (0, 0)),
            pl.BlockSpec((1, 16), lambda b: (0, 0)),
        ],
        out_specs=pl.BlockSpec((G, NCLS), lambda b: (0, 0)),
        out_shape=_f32((G, NCLS)),
        scratch_shapes=[pltpu.VMEM((G, 72), jnp.float32)],
    )(batch3d, agg3, agg3, bo3, W1, b1, W2, b2, W3, b3)


def _pad_edges(src, dst, EP):
    pad = EP - src.shape[0]
    src = jnp.concatenate([src, jnp.zeros((pad,), jnp.int32)])
    dst = jnp.concatenate([dst, jnp.full((pad,), N, jnp.int32)])
    return src, dst


def kernel(x, edge_index, batch, Wl1, bl1, Wr1, br1, att1, bo1, Wl2, bl2,
           Wr2, br2, att2, bo2, Wl3, bl3, Wr3, br3, att3, bo3, W1, b1, W2,
           b2, W3, b3):
    x_p = jnp.pad(x, ((0, NP - N), (0, 0)))
    src1, dst1 = _pad_edges(edge_index[0], edge_index[1], EP1)
    loop = jnp.arange(N, dtype=jnp.int32)
    src2 = jnp.concatenate([edge_index[0], loop])
    dst2 = jnp.concatenate([edge_index[1], loop])
    src2, dst2 = _pad_edges(src2, dst2, EP2)
    batch_p = jnp.concatenate([batch, jnp.full((NP - N,), G, jnp.int32)])
    batch3d = batch_p.reshape(NB, 1, BR)
    Wl3p = jnp.pad(Wl3, ((0, 0), (0, 64)))
    Wr3p = jnp.pad(Wr3, ((0, 0), (0, 64)))
    bl3p = jnp.pad(bl3, (0, 64)).reshape(1, -1)
    br3p = jnp.pad(br3, (0, 64)).reshape(1, -1)

    scoresA1 = _sc_edge_scores_cs(EP1)
    denomB1 = _sc_denom_cs(EP1)
    aggC1 = _sc_aggregate_cs(EP1)
    scoresA2 = _sc_edge_scores_cs(EP2)
    denomB2 = _sc_denom_cs(EP2)
    aggC2 = _sc_aggregate_cs(EP2)
    scoresA3 = _sc_edge_scores_l3(EP2)
    denomB3 = _sc_denom_l3(EP2)
    aggC3 = _sc_aggregate_l3(EP2)

    xl1, xr1 = _tc_proj1(x_p, Wl1, Wr1, bl1.reshape(1, -1),
                         br1.reshape(1, -1))
    e1, gm1 = scoresA1(xl1, xr1, src1, dst1, att1)
    den1 = denomB1(e1, gm1, dst1)
    agg1 = aggC1(xl1, src1, dst1, e1, den1, gm1)

    xl2, xr2 = _tc_proj_mid(agg1, bo1.reshape(1, -1), Wl2, Wr2,
                            bl2.reshape(1, -1), br2.reshape(1, -1))
    e2, gm2 = scoresA2(xl2, xr2, src2, dst2, att2)
    den2 = denomB2(e2, gm2, dst2)
    agg2 = aggC2(xl2, src2, dst2, e2, den2, gm2)

    xl3, xr3 = _tc_proj3(agg2, bo2.reshape(1, -1), Wl3p, Wr3p, bl3p, br3p)
    e3, gm3 = scoresA3(xl3, xr3, src2, dst2, att3)
    den3 = denomB3(e3, gm3, dst2)
    agg3 = aggC3(xl3, src2, dst2, e3, den3, gm3)

    out = _tc_pool_mlp(agg3, batch3d, bo3.reshape(1, -1), W1,
                       b1.reshape(1, -1), W2, b2.reshape(1, -1), W3,
                       b3.reshape(1, -1))
    return out
